# Initial kernel scaffold; baseline (speedup 1.0000x reference)
import jax
import jax.numpy as jnp
from jax.experimental import pallas as pl


def kernel(x, edge_index, edge_attr, batch, lin0_W, lin0_b, edge_W1, edge_b1, edge_W2, edge_b2, conv_b, gru_Wih, gru_Whh, gru_bih, gru_bhh, lstm_Wih, lstm_Whh, lstm_bih, lstm_bhh, lin1_W, lin1_b, lin2_W, lin2_b):
    raise NotImplementedError("write your pallas kernel here")


# speedup vs baseline: 1.4354x; 1.4354x over previous
import functools

import jax
import jax.numpy as jnp
from jax import lax
from jax.experimental import pallas as pl
from jax.experimental.pallas import tpu as pltpu
from jax.experimental.pallas import tpu_sc as plsc

_B = 64

_CHUNK = 128
_NW = 32
_ACC_PAD = 10240


def _sc_mesh():
    return plsc.VectorSubcoreMesh(core_axis_name="c", subcore_axis_name="s")


def _gather_body(table_hbm, idx_hbm, out_hbm, idx_v, rows_v, sem):
    c = lax.axis_index("c")
    s = lax.axis_index("s")
    w = s * 2 + c
    nchunks = idx_hbm.shape[0] // _CHUNK
    base_n = nchunks // _NW
    nk = base_n + jnp.where(w < (nchunks - base_n * _NW), 1, 0)

    def body(k, carry):
        chunk = w + _NW * k
        base = chunk * _CHUNK
        pltpu.sync_copy(idx_hbm.at[pl.ds(base, _CHUNK)], idx_v)
        pltpu.async_copy(table_hbm.at[idx_v], rows_v, sem).wait()
        pltpu.sync_copy(rows_v, out_hbm.at[pl.ds(base, _CHUNK)])
        return carry

    lax.fori_loop(0, nk, body, 0)


def _sc_gather(table, idx):
    e = idx.shape[0]
    dh = table.shape[1]
    k = pl.kernel(
        _gather_body,
        out_type=jax.ShapeDtypeStruct((e, dh), jnp.float32),
        mesh=_sc_mesh(),
        scratch_types=[
            pltpu.VMEM((_CHUNK,), jnp.int32),
            pltpu.VMEM((_CHUNK, dh), jnp.float32),
            pltpu.SemaphoreType.DMA,
        ],
        compiler_params=pltpu.CompilerParams(use_tc_tiling_on_sc=False),
    )
    return k(table, idx)


def _scatter_body(with_deg, msg_hbm, idx_hbm, *refs):
    if with_deg:
        (out_hbm, deg_hbm, acc_sh, deg_sh, idx_v, rows_v, fill_v) = refs
    else:
        (out_hbm, acc_sh, idx_v, rows_v, fill_v) = refs
        deg_hbm = deg_sh = None
    c = lax.axis_index("c")
    s = lax.axis_index("s")
    dh = rows_v.shape[1]
    stripe = _ACC_PAD // 16

    def zb(i, carry):
        fill_v[i, :] = jnp.zeros((16,), jnp.float32)
        return carry

    lax.fori_loop(0, _CHUNK, zb, 0)
    for j in range(stripe // _CHUNK):
        off = s * stripe + j * _CHUNK
        pltpu.sync_copy(fill_v, acc_sh.at[pl.ds(off, _CHUNK)])
        if with_deg:
            pltpu.sync_copy(fill_v, deg_sh.at[pl.ds(off, _CHUNK)])
    plsc.subcore_barrier()

    nchunks = idx_hbm.shape[0] // _CHUNK
    half = nchunks // 2
    base_n = half // 16
    nk = base_n + jnp.where(s < (half - base_n * 16), 1, 0)

    def body(k, carry):
        chunk = c * half + s + 16 * k
        base = chunk * _CHUNK
        pltpu.sync_copy(idx_hbm.at[pl.ds(base, _CHUNK)], idx_v)
        pltpu.sync_copy(msg_hbm.at[pl.ds(base, _CHUNK)], rows_v)
        pltpu.sync_copy(rows_v, acc_sh.at[idx_v], add=True)
        return carry

    lax.fori_loop(0, nk, body, 0)

    if with_deg:
        def ob(i, carry):
            fill_v[i, :] = jnp.ones((16,), jnp.float32)
            return carry

        lax.fori_loop(0, _CHUNK, ob, 0)

        def dbody(k, carry):
            chunk = c * half + s + 16 * k
            base = chunk * _CHUNK
            pltpu.sync_copy(idx_hbm.at[pl.ds(base, _CHUNK)], idx_v)
            pltpu.sync_copy(fill_v, deg_sh.at[idx_v], add=True)
            return carry

        lax.fori_loop(0, nk, dbody, 0)
    plsc.subcore_barrier()

    n = out_hbm.shape[1]
    lo = s * stripe
    full = n // stripe
    tail = n - full * stripe

    @pl.when(s < full)
    def _():
        pltpu.sync_copy(acc_sh.at[pl.ds(lo, stripe)], out_hbm.at[c, pl.ds(lo, stripe)])
        if with_deg:
            pltpu.sync_copy(deg_sh.at[pl.ds(lo, stripe)], deg_hbm.at[c, pl.ds(lo, stripe)])

    if tail:
        @pl.when(s == full)
        def _():
            pltpu.sync_copy(acc_sh.at[pl.ds(full * stripe, tail)],
                            out_hbm.at[c, pl.ds(full * stripe, tail)])
            if with_deg:
                pltpu.sync_copy(deg_sh.at[pl.ds(full * stripe, tail)],
                                deg_hbm.at[c, pl.ds(full * stripe, tail)])


def _sc_scatter(msg, idx, n, with_deg):
    dh = msg.shape[1]
    out_t = jax.ShapeDtypeStruct((2, n, dh), jnp.float32)
    scratch = [pltpu.VMEM_SHARED((_ACC_PAD, dh), jnp.float32)]
    if with_deg:
        scratch.append(pltpu.VMEM_SHARED((_ACC_PAD, dh), jnp.float32))
    scratch += [
        pltpu.VMEM((_CHUNK,), jnp.int32),
        pltpu.VMEM((_CHUNK, dh), jnp.float32),
        pltpu.VMEM((_CHUNK, dh), jnp.float32),
    ]
    k = pl.kernel(
        functools.partial(_scatter_body, with_deg),
        out_type=(out_t, out_t) if with_deg else out_t,
        mesh=_sc_mesh(),
        scratch_types=scratch,
        compiler_params=pltpu.CompilerParams(use_tc_tiling_on_sc=False),
    )
    return k(msg, idx)


def _lin0_body(x_ref, w_ref, b_ref, o_ref):
    o_ref[...] = jax.nn.relu(
        jnp.dot(x_ref[...], w_ref[...], preferred_element_type=jnp.float32,
                precision=lax.Precision.HIGHEST) + b_ref[...])


def _lin0(x, w, b):
    n, din = x.shape
    dh = w.shape[1]
    blk = 1000
    return pl.pallas_call(
        _lin0_body,
        grid=(n // blk,),
        in_specs=[
            pl.BlockSpec((blk, din), lambda i: (i, 0)),
            pl.BlockSpec((din, dh), lambda i: (0, 0)),
            pl.BlockSpec((1, dh), lambda i: (0, 0)),
        ],
        out_specs=pl.BlockSpec((blk, dh), lambda i: (i, 0)),
        out_shape=jax.ShapeDtypeStruct((n, dh), jnp.float32),
    )(x, w, b)


def _edge_body(ea_ref, os_ref, w1_ref, b1_ref, w2_ref, b2_ref, q_ref, r_ref,
               msg_ref):
    hi = lax.Precision.HIGHEST
    eh = jax.nn.relu(
        jnp.dot(ea_ref[...], w1_ref[...], preferred_element_type=jnp.float32,
                precision=hi) + b1_ref[...])
    we = jnp.dot(eh, w2_ref[...], preferred_element_type=jnp.float32,
                 precision=hi) + b2_ref[...]
    rep = jnp.dot(os_ref[...], q_ref[...], preferred_element_type=jnp.float32,
                  precision=hi)
    msg_ref[...] = jnp.dot(we * rep, r_ref[...],
                           preferred_element_type=jnp.float32, precision=hi)


def _edge_msgs(ea, out_src, w1, b1, w2, b2, qm, rm):
    e, de = ea.shape
    dh = out_src.shape[1]
    eh = w1.shape[1]
    d2 = w2.shape[1]
    blk = 3200
    return pl.pallas_call(
        _edge_body,
        grid=(e // blk,),
        in_specs=[
            pl.BlockSpec((blk, de), lambda i: (i, 0)),
            pl.BlockSpec((blk, dh), lambda i: (i, 0)),
            pl.BlockSpec((de, eh), lambda i: (0, 0)),
            pl.BlockSpec((1, eh), lambda i: (0, 0)),
            pl.BlockSpec((eh, d2), lambda i: (0, 0)),
            pl.BlockSpec((1, d2), lambda i: (0, 0)),
            pl.BlockSpec((dh, d2), lambda i: (0, 0)),
            pl.BlockSpec((d2, dh), lambda i: (0, 0)),
        ],
        out_specs=pl.BlockSpec((blk, dh), lambda i: (i, 0)),
        out_shape=jax.ShapeDtypeStruct((e, dh), jnp.float32),
    )(ea, out_src, w1, b1, w2, b2, qm, rm)


def _gru_body(p_ref, dp_ref, cb_ref, h_ref, wih_ref, whh_ref, bih_ref,
              bhh_ref, o_ref):
    hi = lax.Precision.HIGHEST
    dh = h_ref.shape[1]
    agg = p_ref[0] + p_ref[1]
    deg = jnp.maximum(dp_ref[0] + dp_ref[1], 1.0)
    m = jax.nn.relu(agg / deg + cb_ref[...])
    gi = jnp.dot(m, wih_ref[...], preferred_element_type=jnp.float32,
                 precision=hi) + bih_ref[...]
    h = h_ref[...]
    gh = jnp.dot(h, whh_ref[...], preferred_element_type=jnp.float32,
                 precision=hi) + bhh_ref[...]
    r = jax.nn.sigmoid(gi[:, :dh] + gh[:, :dh])
    z = jax.nn.sigmoid(gi[:, dh:2 * dh] + gh[:, dh:2 * dh])
    nn = jnp.tanh(gi[:, 2 * dh:] + r * gh[:, 2 * dh:])
    o_ref[...] = (1.0 - z) * nn + z * h


def _gru(partials, degp, cb, h, wih, whh, bih, bhh):
    n, dh = h.shape
    d3 = wih.shape[1]
    blk = 2000
    return pl.pallas_call(
        _gru_body,
        grid=(n // blk,),
        in_specs=[
            pl.BlockSpec((2, blk, dh), lambda i: (0, i, 0)),
            pl.BlockSpec((2, blk, dh), lambda i: (0, i, 0)),
            pl.BlockSpec((1, dh), lambda i: (0, 0)),
            pl.BlockSpec((blk, dh), lambda i: (i, 0)),
            pl.BlockSpec((dh, d3), lambda i: (0, 0)),
            pl.BlockSpec((dh, d3), lambda i: (0, 0)),
            pl.BlockSpec((1, d3), lambda i: (0, 0)),
            pl.BlockSpec((1, d3), lambda i: (0, 0)),
        ],
        out_specs=pl.BlockSpec((blk, dh), lambda i: (i, 0)),
        out_shape=jax.ShapeDtypeStruct((n, dh), jnp.float32),
    )(partials, degp, cb, h, wih, whh, bih, bhh)


def _s2s_body(out_ref, batch_ref, wih_ref, whh_ref, bih_ref, bhh_ref,
              l1w_ref, l1b_ref, l2w_ref, l2b_ref, y_ref):
    hi = lax.Precision.HIGHEST
    xo = out_ref[...]
    n, dh = xo.shape
    rows = lax.broadcasted_iota(jnp.int32, (_B, n), 0)
    mb = jnp.broadcast_to(batch_ref[...], (_B, n))
    mask = rows == mb

    q_star = jnp.zeros((_B, 2 * dh), jnp.float32)
    hh = jnp.zeros((_B, dh), jnp.float32)
    cc = jnp.zeros((_B, dh), jnp.float32)
    for _ in range(3):
        g = (jnp.dot(q_star, wih_ref[...], preferred_element_type=jnp.float32,
                     precision=hi) + bih_ref[...] +
             jnp.dot(hh, whh_ref[...], preferred_element_type=jnp.float32,
                     precision=hi) + bhh_ref[...])
        ig = jax.nn.sigmoid(g[:, :dh])
        fg = jax.nn.sigmoid(g[:, dh:2 * dh])
        cg = jnp.tanh(g[:, 2 * dh:3 * dh])
        og = jax.nn.sigmoid(g[:, 3 * dh:])
        cc = fg * cc + ig * cg
        hh = og * jnp.tanh(cc)
        e_all = lax.dot_general(hh, xo, (((1,), (1,)), ((), ())),
                                preferred_element_type=jnp.float32,
                                precision=hi)
        e_masked = jnp.where(mask, e_all, -jnp.inf)
        emax = jnp.max(e_masked, axis=1, keepdims=True)
        emax = jnp.where(jnp.isfinite(emax), emax, 0.0)
        a = jnp.where(mask, jnp.exp(e_all - emax), 0.0)
        denom = jnp.sum(a, axis=1, keepdims=True)
        a = a / jnp.maximum(denom, 1e-30)
        r_read = jnp.dot(a, xo, preferred_element_type=jnp.float32,
                         precision=hi)
        q_star = jnp.concatenate([hh, r_read], axis=1)
    y = jax.nn.relu(
        jnp.dot(q_star, l1w_ref[...], preferred_element_type=jnp.float32,
                precision=hi) + l1b_ref[...])
    y_ref[...] = jnp.dot(y, l2w_ref[...], preferred_element_type=jnp.float32,
                         precision=hi) + l2b_ref[...]


def _s2s(out, batch2d, wih, whh, bih, bhh, l1w, l1b, l2w, l2b):
    dout = l2w.shape[1]
    return pl.pallas_call(
        _s2s_body,
        out_shape=jax.ShapeDtypeStruct((_B, dout), jnp.float32),
    )(out, batch2d, wih, whh, bih, bhh, l1w, l1b, l2w, l2b)


def kernel(x, edge_index, edge_attr, batch, lin0_W, lin0_b, edge_W1, edge_b1,
           edge_W2, edge_b2, conv_b, gru_Wih, gru_Whh, gru_bih, gru_bhh,
           lstm_Wih, lstm_Whh, lstm_bih, lstm_bhh, lin1_W, lin1_b, lin2_W,
           lin2_b):
    n = x.shape[0]
    dh = lin0_W.shape[1]
    d2 = dh * dh
    nlayers = edge_W1.shape[0]
    src = edge_index[0]
    dst = edge_index[1]

    j = jnp.arange(d2, dtype=jnp.int32)
    qm = (j[None, :] // dh == jnp.arange(dh, dtype=jnp.int32)[:, None]
          ).astype(jnp.float32)
    rm = (j[:, None] % dh == jnp.arange(dh, dtype=jnp.int32)[None, :]
          ).astype(jnp.float32)

    h = _lin0(x, lin0_W, lin0_b.reshape(1, -1))
    degp = None
    for li in range(nlayers):
        out_src = _sc_gather(h, src)
        msg = _edge_msgs(edge_attr, out_src, edge_W1[li],
                         edge_b1[li].reshape(1, -1), edge_W2[li],
                         edge_b2[li].reshape(1, -1), qm, rm)
        if degp is None:
            partials, degp = _sc_scatter(msg, dst, n, True)
        else:
            partials = _sc_scatter(msg, dst, n, False)
        h = _gru(partials, degp, conv_b[li].reshape(1, -1), h, gru_Wih,
                 gru_Whh, gru_bih.reshape(1, -1), gru_bhh.reshape(1, -1))
    return _s2s(h, batch.reshape(1, -1), lstm_Wih, lstm_Whh,
                lstm_bih.reshape(1, -1), lstm_bhh.reshape(1, -1), lin1_W,
                lin1_b.reshape(1, -1), lin2_W, lin2_b.reshape(1, -1))


# speedup vs baseline: 3.0884x; 2.1515x over previous
import functools

import jax
import jax.numpy as jnp
from jax import lax
from jax.experimental import pallas as pl
from jax.experimental.pallas import tpu as pltpu
from jax.experimental.pallas import tpu_sc as plsc

_B = 64

_CHUNK = 128
_NW = 32
_ACC_PAD = 10240


def _sc_mesh():
    return plsc.VectorSubcoreMesh(core_axis_name="c", subcore_axis_name="s")


def _gather_body(table_hbm, idx_hbm, out_hbm, idx_v, rows_v, sem):
    c = lax.axis_index("c")
    s = lax.axis_index("s")
    w = s * 2 + c
    nchunks = idx_hbm.shape[0] // _CHUNK
    base_n = nchunks // _NW
    nk = base_n + jnp.where(w < (nchunks - base_n * _NW), 1, 0)

    def body(k, carry):
        chunk = w + _NW * k
        base = chunk * _CHUNK
        pltpu.sync_copy(idx_hbm.at[pl.ds(base, _CHUNK)], idx_v)
        pltpu.async_copy(table_hbm.at[idx_v], rows_v, sem).wait()
        pltpu.sync_copy(rows_v, out_hbm.at[pl.ds(base, _CHUNK)])
        return carry

    lax.fori_loop(0, nk, body, 0)


def _sc_gather(table, idx):
    e = idx.shape[0]
    dh = table.shape[1]
    k = pl.kernel(
        _gather_body,
        out_type=jax.ShapeDtypeStruct((e, dh), jnp.float32),
        mesh=_sc_mesh(),
        scratch_types=[
            pltpu.VMEM((_CHUNK,), jnp.int32),
            pltpu.VMEM((_CHUNK, dh), jnp.float32),
            pltpu.SemaphoreType.DMA,
        ],
        compiler_params=pltpu.CompilerParams(use_tc_tiling_on_sc=False),
    )
    return k(table, idx)


def _scatter_body(with_deg, msg_hbm, idx_hbm, *refs):
    if with_deg:
        (out_hbm, deg_hbm, acc_sh, deg_sh, idx_v, rows_v, fill_v) = refs
    else:
        (out_hbm, acc_sh, idx_v, rows_v, fill_v) = refs
        deg_hbm = deg_sh = None
    c = lax.axis_index("c")
    s = lax.axis_index("s")
    dh = rows_v.shape[1]
    stripe = _ACC_PAD // 16

    def zb(i, carry):
        fill_v[i, :] = jnp.zeros((16,), jnp.float32)
        return carry

    lax.fori_loop(0, _CHUNK, zb, 0)
    for j in range(stripe // _CHUNK):
        off = s * stripe + j * _CHUNK
        pltpu.sync_copy(fill_v, acc_sh.at[pl.ds(off, _CHUNK)])
        if with_deg:
            pltpu.sync_copy(fill_v, deg_sh.at[pl.ds(off, _CHUNK)])
    plsc.subcore_barrier()

    nchunks = idx_hbm.shape[0] // _CHUNK
    half = nchunks // 2
    base_n = half // 16
    nk = base_n + jnp.where(s < (half - base_n * 16), 1, 0)

    def body(k, carry):
        chunk = c * half + s + 16 * k
        base = chunk * _CHUNK
        pltpu.sync_copy(idx_hbm.at[pl.ds(base, _CHUNK)], idx_v)
        pltpu.sync_copy(msg_hbm.at[pl.ds(base, _CHUNK)], rows_v)
        pltpu.sync_copy(rows_v, acc_sh.at[idx_v], add=True)
        return carry

    lax.fori_loop(0, nk, body, 0)

    if with_deg:
        def ob(i, carry):
            fill_v[i, :] = jnp.ones((16,), jnp.float32)
            return carry

        lax.fori_loop(0, _CHUNK, ob, 0)

        def dbody(k, carry):
            chunk = c * half + s + 16 * k
            base = chunk * _CHUNK
            pltpu.sync_copy(idx_hbm.at[pl.ds(base, _CHUNK)], idx_v)
            pltpu.sync_copy(fill_v, deg_sh.at[idx_v], add=True)
            return carry

        lax.fori_loop(0, nk, dbody, 0)
    plsc.subcore_barrier()

    n = out_hbm.shape[1]
    lo = s * stripe
    full = n // stripe
    tail = n - full * stripe

    @pl.when(s < full)
    def _():
        pltpu.sync_copy(acc_sh.at[pl.ds(lo, stripe)], out_hbm.at[c, pl.ds(lo, stripe)])
        if with_deg:
            pltpu.sync_copy(deg_sh.at[pl.ds(lo, stripe)], deg_hbm.at[c, pl.ds(lo, stripe)])

    if tail:
        @pl.when(s == full)
        def _():
            pltpu.sync_copy(acc_sh.at[pl.ds(full * stripe, tail)],
                            out_hbm.at[c, pl.ds(full * stripe, tail)])
            if with_deg:
                pltpu.sync_copy(deg_sh.at[pl.ds(full * stripe, tail)],
                                deg_hbm.at[c, pl.ds(full * stripe, tail)])


def _sc_scatter(msg, idx, n, with_deg):
    dh = msg.shape[1]
    out_t = jax.ShapeDtypeStruct((2, n, dh), jnp.float32)
    scratch = [pltpu.VMEM_SHARED((_ACC_PAD, dh), jnp.float32)]
    if with_deg:
        scratch.append(pltpu.VMEM_SHARED((_ACC_PAD, dh), jnp.float32))
    scratch += [
        pltpu.VMEM((_CHUNK,), jnp.int32),
        pltpu.VMEM((_CHUNK, dh), jnp.float32),
        pltpu.VMEM((_CHUNK, dh), jnp.float32),
    ]
    k = pl.kernel(
        functools.partial(_scatter_body, with_deg),
        out_type=(out_t, out_t) if with_deg else out_t,
        mesh=_sc_mesh(),
        scratch_types=scratch,
        compiler_params=pltpu.CompilerParams(use_tc_tiling_on_sc=False),
    )
    return k(msg, idx)


def _lin0_body(x_ref, w_ref, b_ref, o_ref):
    o_ref[...] = jax.nn.relu(
        jnp.dot(x_ref[...], w_ref[...], preferred_element_type=jnp.float32,
                precision=lax.Precision.HIGHEST) + b_ref[...])


def _lin0(x, w, b):
    n, din = x.shape
    dh = w.shape[1]
    blk = 1000
    return pl.pallas_call(
        _lin0_body,
        grid=(n // blk,),
        in_specs=[
            pl.BlockSpec((blk, din), lambda i: (i, 0)),
            pl.BlockSpec((din, dh), lambda i: (0, 0)),
            pl.BlockSpec((1, dh), lambda i: (0, 0)),
        ],
        out_specs=pl.BlockSpec((blk, dh), lambda i: (i, 0)),
        out_shape=jax.ShapeDtypeStruct((n, dh), jnp.float32),
    )(x, w, b)


def _edge_body(ea_ref, os_ref, w1_ref, b1_ref, w2_ref, b2_ref, r_ref, msg_ref):
    bf = jnp.bfloat16
    dh = os_ref.shape[1]
    eh = jax.nn.relu(
        jnp.dot(ea_ref[...].astype(bf), w1_ref[...].astype(bf),
                preferred_element_type=jnp.float32) + b1_ref[...])
    we = jnp.dot(eh.astype(bf), w2_ref[...].astype(bf),
                 preferred_element_type=jnp.float32) + b2_ref[...]
    rep = jnp.tile(os_ref[...], (1, dh))
    msg_ref[...] = jnp.dot((we * rep).astype(bf), r_ref[...].astype(bf),
                           preferred_element_type=jnp.float32)


def _edge_msgs(ea, out_src, w1, b1, w2, b2, rm):
    e, de = ea.shape
    dh = out_src.shape[1]
    eh = w1.shape[1]
    d2 = w2.shape[1]
    blk = 3200
    return pl.pallas_call(
        _edge_body,
        grid=(e // blk,),
        in_specs=[
            pl.BlockSpec((blk, de), lambda i: (i, 0)),
            pl.BlockSpec((blk, dh), lambda i: (i, 0)),
            pl.BlockSpec((de, eh), lambda i: (0, 0)),
            pl.BlockSpec((1, eh), lambda i: (0, 0)),
            pl.BlockSpec((eh, d2), lambda i: (0, 0)),
            pl.BlockSpec((1, d2), lambda i: (0, 0)),
            pl.BlockSpec((d2, dh), lambda i: (0, 0)),
        ],
        out_specs=pl.BlockSpec((blk, dh), lambda i: (i, 0)),
        out_shape=jax.ShapeDtypeStruct((e, dh), jnp.float32),
    )(ea, out_src, w1, b1, w2, b2, rm)


def _gru_body(p_ref, dp_ref, cb_ref, h_ref, wih_ref, whh_ref, bih_ref,
              bhh_ref, o_ref):
    hi = lax.Precision.HIGHEST
    dh = h_ref.shape[1]
    agg = p_ref[0] + p_ref[1]
    deg = jnp.maximum(dp_ref[0] + dp_ref[1], 1.0)
    m = jax.nn.relu(agg / deg + cb_ref[...])
    gi = jnp.dot(m, wih_ref[...], preferred_element_type=jnp.float32,
                 precision=hi) + bih_ref[...]
    h = h_ref[...]
    gh = jnp.dot(h, whh_ref[...], preferred_element_type=jnp.float32,
                 precision=hi) + bhh_ref[...]
    r = jax.nn.sigmoid(gi[:, :dh] + gh[:, :dh])
    z = jax.nn.sigmoid(gi[:, dh:2 * dh] + gh[:, dh:2 * dh])
    nn = jnp.tanh(gi[:, 2 * dh:] + r * gh[:, 2 * dh:])
    o_ref[...] = (1.0 - z) * nn + z * h


def _gru(partials, degp, cb, h, wih, whh, bih, bhh):
    n, dh = h.shape
    d3 = wih.shape[1]
    blk = 2000
    return pl.pallas_call(
        _gru_body,
        grid=(n // blk,),
        in_specs=[
            pl.BlockSpec((2, blk, dh), lambda i: (0, i, 0)),
            pl.BlockSpec((2, blk, dh), lambda i: (0, i, 0)),
            pl.BlockSpec((1, dh), lambda i: (0, 0)),
            pl.BlockSpec((blk, dh), lambda i: (i, 0)),
            pl.BlockSpec((dh, d3), lambda i: (0, 0)),
            pl.BlockSpec((dh, d3), lambda i: (0, 0)),
            pl.BlockSpec((1, d3), lambda i: (0, 0)),
            pl.BlockSpec((1, d3), lambda i: (0, 0)),
        ],
        out_specs=pl.BlockSpec((blk, dh), lambda i: (i, 0)),
        out_shape=jax.ShapeDtypeStruct((n, dh), jnp.float32),
    )(partials, degp, cb, h, wih, whh, bih, bhh)


def _s2s_body(out_ref, batch_ref, wih_ref, whh_ref, bih_ref, bhh_ref,
              l1w_ref, l1b_ref, l2w_ref, l2b_ref, y_ref):
    hi = lax.Precision.HIGHEST
    xo = out_ref[...]
    n, dh = xo.shape
    rows = lax.broadcasted_iota(jnp.int32, (_B, n), 0)
    mb = jnp.broadcast_to(batch_ref[...], (_B, n))
    mask = rows == mb

    q_star = jnp.zeros((_B, 2 * dh), jnp.float32)
    hh = jnp.zeros((_B, dh), jnp.float32)
    cc = jnp.zeros((_B, dh), jnp.float32)
    for _ in range(3):
        g = (jnp.dot(q_star, wih_ref[...], preferred_element_type=jnp.float32,
                     precision=hi) + bih_ref[...] +
             jnp.dot(hh, whh_ref[...], preferred_element_type=jnp.float32,
                     precision=hi) + bhh_ref[...])
        ig = jax.nn.sigmoid(g[:, :dh])
        fg = jax.nn.sigmoid(g[:, dh:2 * dh])
        cg = jnp.tanh(g[:, 2 * dh:3 * dh])
        og = jax.nn.sigmoid(g[:, 3 * dh:])
        cc = fg * cc + ig * cg
        hh = og * jnp.tanh(cc)
        e_all = lax.dot_general(hh, xo, (((1,), (1,)), ((), ())),
                                preferred_element_type=jnp.float32,
                                precision=hi)
        e_masked = jnp.where(mask, e_all, -jnp.inf)
        emax = jnp.max(e_masked, axis=1, keepdims=True)
        emax = jnp.where(jnp.isfinite(emax), emax, 0.0)
        a = jnp.where(mask, jnp.exp(e_all - emax), 0.0)
        denom = jnp.sum(a, axis=1, keepdims=True)
        a = a / jnp.maximum(denom, 1e-30)
        r_read = jnp.dot(a, xo, preferred_element_type=jnp.float32,
                         precision=hi)
        q_star = jnp.concatenate([hh, r_read], axis=1)
    y = jax.nn.relu(
        jnp.dot(q_star, l1w_ref[...], preferred_element_type=jnp.float32,
                precision=hi) + l1b_ref[...])
    y_ref[...] = jnp.dot(y, l2w_ref[...], preferred_element_type=jnp.float32,
                         precision=hi) + l2b_ref[...]


def _s2s(out, batch2d, wih, whh, bih, bhh, l1w, l1b, l2w, l2b):
    dout = l2w.shape[1]
    return pl.pallas_call(
        _s2s_body,
        out_shape=jax.ShapeDtypeStruct((_B, dout), jnp.float32),
    )(out, batch2d, wih, whh, bih, bhh, l1w, l1b, l2w, l2b)


def kernel(x, edge_index, edge_attr, batch, lin0_W, lin0_b, edge_W1, edge_b1,
           edge_W2, edge_b2, conv_b, gru_Wih, gru_Whh, gru_bih, gru_bhh,
           lstm_Wih, lstm_Whh, lstm_bih, lstm_bhh, lin1_W, lin1_b, lin2_W,
           lin2_b):
    n = x.shape[0]
    dh = lin0_W.shape[1]
    d2 = dh * dh
    nlayers = edge_W1.shape[0]
    src = edge_index[0]
    dst = edge_index[1]

    j = jnp.arange(d2, dtype=jnp.int32)
    rm = (j[:, None] // dh == jnp.arange(dh, dtype=jnp.int32)[None, :]
          ).astype(jnp.float32)
    w2p = edge_W2.reshape(nlayers, -1, dh, dh).transpose(0, 1, 3, 2).reshape(
        nlayers, -1, d2)
    b2p = edge_b2.reshape(nlayers, dh, dh).transpose(0, 2, 1).reshape(
        nlayers, d2)

    h = _lin0(x, lin0_W, lin0_b.reshape(1, -1))
    degp = None
    for li in range(nlayers):
        out_src = _sc_gather(h, src)
        msg = _edge_msgs(edge_attr, out_src, edge_W1[li],
                         edge_b1[li].reshape(1, -1), w2p[li],
                         b2p[li].reshape(1, -1), rm)
        if degp is None:
            partials, degp = _sc_scatter(msg, dst, n, True)
        else:
            partials = _sc_scatter(msg, dst, n, False)
        h = _gru(partials, degp, conv_b[li].reshape(1, -1), h, gru_Wih,
                 gru_Whh, gru_bih.reshape(1, -1), gru_bhh.reshape(1, -1))
    return _s2s(h, batch.reshape(1, -1), lstm_Wih, lstm_Whh,
                lstm_bih.reshape(1, -1), lstm_bhh.reshape(1, -1), lin1_W,
                lin1_b.reshape(1, -1), lin2_W, lin2_b.reshape(1, -1))


# speedup vs baseline: 3.4509x; 1.1174x over previous
import functools

import jax
import jax.numpy as jnp
from jax import lax
from jax.experimental import pallas as pl
from jax.experimental.pallas import tpu as pltpu
from jax.experimental.pallas import tpu_sc as plsc

_B = 64

_CHUNK = 128
_NW = 32
_ACC_PAD = 10240


def _sc_mesh():
    return plsc.VectorSubcoreMesh(core_axis_name="c", subcore_axis_name="s")


def _gather_body(table_hbm, idx_hbm, out_hbm, idx_v, rows_v, sem):
    c = lax.axis_index("c")
    s = lax.axis_index("s")
    w = s * 2 + c
    nchunks = idx_hbm.shape[0] // _CHUNK
    base_n = nchunks // _NW
    nk = base_n + jnp.where(w < (nchunks - base_n * _NW), 1, 0)

    def body(k, carry):
        chunk = w + _NW * k
        base = chunk * _CHUNK
        pltpu.sync_copy(idx_hbm.at[pl.ds(base, _CHUNK)], idx_v)
        pltpu.async_copy(table_hbm.at[idx_v], rows_v, sem).wait()
        pltpu.sync_copy(rows_v, out_hbm.at[pl.ds(base, _CHUNK)])
        return carry

    lax.fori_loop(0, nk, body, 0)


def _sc_gather(table, idx):
    e = idx.shape[0]
    dh = table.shape[1]
    k = pl.kernel(
        _gather_body,
        out_type=jax.ShapeDtypeStruct((e, dh), jnp.float32),
        mesh=_sc_mesh(),
        scratch_types=[
            pltpu.VMEM((_CHUNK,), jnp.int32),
            pltpu.VMEM((_CHUNK, dh), jnp.float32),
            pltpu.SemaphoreType.DMA,
        ],
        compiler_params=pltpu.CompilerParams(use_tc_tiling_on_sc=False),
    )
    return k(table, idx)


def _scatter_body(with_deg, msg_hbm, idx_hbm, *refs):
    if with_deg:
        (out_hbm, deg_hbm, acc_sh, deg_sh, idx_v, rows_v, fill_v) = refs
    else:
        (out_hbm, acc_sh, idx_v, rows_v, fill_v) = refs
        deg_hbm = deg_sh = None
    c = lax.axis_index("c")
    s = lax.axis_index("s")
    dh = rows_v.shape[1]
    stripe = _ACC_PAD // 16

    def zb(i, carry):
        fill_v[i, :] = jnp.zeros((16,), jnp.float32)
        return carry

    lax.fori_loop(0, _CHUNK, zb, 0)
    for j in range(stripe // _CHUNK):
        off = s * stripe + j * _CHUNK
        pltpu.sync_copy(fill_v, acc_sh.at[pl.ds(off, _CHUNK)])
        if with_deg:
            pltpu.sync_copy(fill_v, deg_sh.at[pl.ds(off, _CHUNK)])
    plsc.subcore_barrier()

    nchunks = idx_hbm.shape[0] // _CHUNK
    half = nchunks // 2
    base_n = half // 16
    nk = base_n + jnp.where(s < (half - base_n * 16), 1, 0)

    def body(k, carry):
        chunk = c * half + s + 16 * k
        base = chunk * _CHUNK
        pltpu.sync_copy(idx_hbm.at[pl.ds(base, _CHUNK)], idx_v)
        pltpu.sync_copy(msg_hbm.at[pl.ds(base, _CHUNK)], rows_v)
        pltpu.sync_copy(rows_v, acc_sh.at[idx_v], add=True)
        return carry

    lax.fori_loop(0, nk, body, 0)

    if with_deg:
        def ob(i, carry):
            fill_v[i, :] = jnp.ones((16,), jnp.float32)
            return carry

        lax.fori_loop(0, _CHUNK, ob, 0)

        def dbody(k, carry):
            chunk = c * half + s + 16 * k
            base = chunk * _CHUNK
            pltpu.sync_copy(idx_hbm.at[pl.ds(base, _CHUNK)], idx_v)
            pltpu.sync_copy(fill_v, deg_sh.at[idx_v], add=True)
            return carry

        lax.fori_loop(0, nk, dbody, 0)
    plsc.subcore_barrier()

    n = out_hbm.shape[1]
    lo = s * stripe
    full = n // stripe
    tail = n - full * stripe

    @pl.when(s < full)
    def _():
        pltpu.sync_copy(acc_sh.at[pl.ds(lo, stripe)], out_hbm.at[c, pl.ds(lo, stripe)])
        if with_deg:
            pltpu.sync_copy(deg_sh.at[pl.ds(lo, stripe)], deg_hbm.at[c, pl.ds(lo, stripe)])

    if tail:
        @pl.when(s == full)
        def _():
            pltpu.sync_copy(acc_sh.at[pl.ds(full * stripe, tail)],
                            out_hbm.at[c, pl.ds(full * stripe, tail)])
            if with_deg:
                pltpu.sync_copy(deg_sh.at[pl.ds(full * stripe, tail)],
                                deg_hbm.at[c, pl.ds(full * stripe, tail)])


def _sc_scatter(msg, idx, n, with_deg):
    dh = msg.shape[1]
    out_t = jax.ShapeDtypeStruct((2, n, dh), jnp.float32)
    scratch = [pltpu.VMEM_SHARED((_ACC_PAD, dh), jnp.float32)]
    if with_deg:
        scratch.append(pltpu.VMEM_SHARED((_ACC_PAD, dh), jnp.float32))
    scratch += [
        pltpu.VMEM((_CHUNK,), jnp.int32),
        pltpu.VMEM((_CHUNK, dh), jnp.float32),
        pltpu.VMEM((_CHUNK, dh), jnp.float32),
    ]
    k = pl.kernel(
        functools.partial(_scatter_body, with_deg),
        out_type=(out_t, out_t) if with_deg else out_t,
        mesh=_sc_mesh(),
        scratch_types=scratch,
        compiler_params=pltpu.CompilerParams(use_tc_tiling_on_sc=False),
    )
    return k(msg, idx)


def _lin0_body(x_ref, w_ref, b_ref, o_ref):
    o_ref[...] = jax.nn.relu(
        jnp.dot(x_ref[...], w_ref[...], preferred_element_type=jnp.float32,
                precision=lax.Precision.HIGHEST) + b_ref[...])


def _lin0(x, w, b):
    n, din = x.shape
    dh = w.shape[1]
    blk = 1000
    return pl.pallas_call(
        _lin0_body,
        grid=(n // blk,),
        in_specs=[
            pl.BlockSpec((blk, din), lambda i: (i, 0)),
            pl.BlockSpec((din, dh), lambda i: (0, 0)),
            pl.BlockSpec((1, dh), lambda i: (0, 0)),
        ],
        out_specs=pl.BlockSpec((blk, dh), lambda i: (i, 0)),
        out_shape=jax.ShapeDtypeStruct((n, dh), jnp.float32),
    )(x, w, b)


def _edge_body(ea_ref, os_ref, w1_ref, b1_ref, w2_ref, b2_ref, q_ref, r_ref,
               msg_ref):
    bf = jnp.bfloat16
    eh = jax.nn.relu(
        jnp.dot(ea_ref[...].astype(bf), w1_ref[...].astype(bf),
                preferred_element_type=jnp.float32) + b1_ref[...])
    we = jnp.dot(eh.astype(bf), w2_ref[...].astype(bf),
                 preferred_element_type=jnp.float32) + b2_ref[...]
    rep = jnp.dot(os_ref[...].astype(bf), q_ref[...],
                  preferred_element_type=jnp.float32)
    msg_ref[...] = jnp.dot((we * rep).astype(bf), r_ref[...],
                           preferred_element_type=jnp.float32)


def _edge_msgs(ea, out_src, w1, b1, w2, b2, qm, rm):
    e, de = ea.shape
    dh = out_src.shape[1]
    eh = w1.shape[1]
    d2 = w2.shape[1]
    blk = 3200
    return pl.pallas_call(
        _edge_body,
        grid=(e // blk,),
        in_specs=[
            pl.BlockSpec((blk, de), lambda i: (i, 0)),
            pl.BlockSpec((blk, dh), lambda i: (i, 0)),
            pl.BlockSpec((de, eh), lambda i: (0, 0)),
            pl.BlockSpec((1, eh), lambda i: (0, 0)),
            pl.BlockSpec((eh, d2), lambda i: (0, 0)),
            pl.BlockSpec((1, d2), lambda i: (0, 0)),
            pl.BlockSpec((dh, d2), lambda i: (0, 0)),
            pl.BlockSpec((d2, dh), lambda i: (0, 0)),
        ],
        out_specs=pl.BlockSpec((blk, dh), lambda i: (i, 0)),
        out_shape=jax.ShapeDtypeStruct((e, dh), jnp.float32),
    )(ea, out_src, w1, b1, w2, b2, qm, rm)


def _gru_body(p_ref, dp_ref, cb_ref, h_ref, wih_ref, whh_ref, bih_ref,
              bhh_ref, o_ref):
    hi = lax.Precision.HIGHEST
    dh = h_ref.shape[1]
    agg = p_ref[0] + p_ref[1]
    deg = jnp.maximum(dp_ref[0] + dp_ref[1], 1.0)
    m = jax.nn.relu(agg / deg + cb_ref[...])
    gi = jnp.dot(m, wih_ref[...], preferred_element_type=jnp.float32,
                 precision=hi) + bih_ref[...]
    h = h_ref[...]
    gh = jnp.dot(h, whh_ref[...], preferred_element_type=jnp.float32,
                 precision=hi) + bhh_ref[...]
    r = jax.nn.sigmoid(gi[:, :dh] + gh[:, :dh])
    z = jax.nn.sigmoid(gi[:, dh:2 * dh] + gh[:, dh:2 * dh])
    nn = jnp.tanh(gi[:, 2 * dh:] + r * gh[:, 2 * dh:])
    o_ref[...] = (1.0 - z) * nn + z * h


def _gru(partials, degp, cb, h, wih, whh, bih, bhh):
    n, dh = h.shape
    d3 = wih.shape[1]
    blk = 2000
    return pl.pallas_call(
        _gru_body,
        grid=(n // blk,),
        in_specs=[
            pl.BlockSpec((2, blk, dh), lambda i: (0, i, 0)),
            pl.BlockSpec((2, blk, dh), lambda i: (0, i, 0)),
            pl.BlockSpec((1, dh), lambda i: (0, 0)),
            pl.BlockSpec((blk, dh), lambda i: (i, 0)),
            pl.BlockSpec((dh, d3), lambda i: (0, 0)),
            pl.BlockSpec((dh, d3), lambda i: (0, 0)),
            pl.BlockSpec((1, d3), lambda i: (0, 0)),
            pl.BlockSpec((1, d3), lambda i: (0, 0)),
        ],
        out_specs=pl.BlockSpec((blk, dh), lambda i: (i, 0)),
        out_shape=jax.ShapeDtypeStruct((n, dh), jnp.float32),
    )(partials, degp, cb, h, wih, whh, bih, bhh)


def _s2s_body(out_ref, batch_ref, wih_ref, whh_ref, bih_ref, bhh_ref,
              l1w_ref, l1b_ref, l2w_ref, l2b_ref, y_ref):
    hi = lax.Precision.HIGHEST
    xo = out_ref[...]
    n, dh = xo.shape
    rows = lax.broadcasted_iota(jnp.int32, (_B, n), 0)
    mb = jnp.broadcast_to(batch_ref[...], (_B, n))
    mask = rows == mb

    q_star = jnp.zeros((_B, 2 * dh), jnp.float32)
    hh = jnp.zeros((_B, dh), jnp.float32)
    cc = jnp.zeros((_B, dh), jnp.float32)
    for _ in range(3):
        g = (jnp.dot(q_star, wih_ref[...], preferred_element_type=jnp.float32,
                     precision=hi) + bih_ref[...] +
             jnp.dot(hh, whh_ref[...], preferred_element_type=jnp.float32,
                     precision=hi) + bhh_ref[...])
        ig = jax.nn.sigmoid(g[:, :dh])
        fg = jax.nn.sigmoid(g[:, dh:2 * dh])
        cg = jnp.tanh(g[:, 2 * dh:3 * dh])
        og = jax.nn.sigmoid(g[:, 3 * dh:])
        cc = fg * cc + ig * cg
        hh = og * jnp.tanh(cc)
        e_all = lax.dot_general(hh, xo, (((1,), (1,)), ((), ())),
                                preferred_element_type=jnp.float32,
                                precision=hi)
        e_masked = jnp.where(mask, e_all, -jnp.inf)
        emax = jnp.max(e_masked, axis=1, keepdims=True)
        emax = jnp.where(jnp.isfinite(emax), emax, 0.0)
        a = jnp.where(mask, jnp.exp(e_all - emax), 0.0)
        denom = jnp.sum(a, axis=1, keepdims=True)
        a = a / jnp.maximum(denom, 1e-30)
        r_read = jnp.dot(a, xo, preferred_element_type=jnp.float32,
                         precision=hi)
        q_star = jnp.concatenate([hh, r_read], axis=1)
    y = jax.nn.relu(
        jnp.dot(q_star, l1w_ref[...], preferred_element_type=jnp.float32,
                precision=hi) + l1b_ref[...])
    y_ref[...] = jnp.dot(y, l2w_ref[...], preferred_element_type=jnp.float32,
                         precision=hi) + l2b_ref[...]


def _s2s(out, batch2d, wih, whh, bih, bhh, l1w, l1b, l2w, l2b):
    dout = l2w.shape[1]
    return pl.pallas_call(
        _s2s_body,
        out_shape=jax.ShapeDtypeStruct((_B, dout), jnp.float32),
    )(out, batch2d, wih, whh, bih, bhh, l1w, l1b, l2w, l2b)


def kernel(x, edge_index, edge_attr, batch, lin0_W, lin0_b, edge_W1, edge_b1,
           edge_W2, edge_b2, conv_b, gru_Wih, gru_Whh, gru_bih, gru_bhh,
           lstm_Wih, lstm_Whh, lstm_bih, lstm_bhh, lin1_W, lin1_b, lin2_W,
           lin2_b):
    n = x.shape[0]
    dh = lin0_W.shape[1]
    d2 = dh * dh
    nlayers = edge_W1.shape[0]
    src = edge_index[0]
    dst = edge_index[1]

    j = jnp.arange(d2, dtype=jnp.int32)
    qm = (j[None, :] // dh == jnp.arange(dh, dtype=jnp.int32)[:, None]
          ).astype(jnp.bfloat16)
    rm = (j[:, None] % dh == jnp.arange(dh, dtype=jnp.int32)[None, :]
          ).astype(jnp.bfloat16)

    h = _lin0(x, lin0_W, lin0_b.reshape(1, -1))
    degp = None
    for li in range(nlayers):
        out_src = _sc_gather(h, src)
        msg = _edge_msgs(edge_attr, out_src, edge_W1[li],
                         edge_b1[li].reshape(1, -1), edge_W2[li],
                         edge_b2[li].reshape(1, -1), qm, rm)
        if degp is None:
            partials, degp = _sc_scatter(msg, dst, n, True)
        else:
            partials = _sc_scatter(msg, dst, n, False)
        h = _gru(partials, degp, conv_b[li].reshape(1, -1), h, gru_Wih,
                 gru_Whh, gru_bih.reshape(1, -1), gru_bhh.reshape(1, -1))
    return _s2s(h, batch.reshape(1, -1), lstm_Wih, lstm_Whh,
                lstm_bih.reshape(1, -1), lstm_bhh.reshape(1, -1), lin1_W,
                lin1_b.reshape(1, -1), lin2_W, lin2_b.reshape(1, -1))


# speedup vs baseline: 3.9823x; 1.1540x over previous
import functools

import jax
import jax.numpy as jnp
from jax import lax
from jax.experimental import pallas as pl
from jax.experimental.pallas import tpu as pltpu
from jax.experimental.pallas import tpu_sc as plsc

_B = 64

_CHUNK = 128
_NW = 32
_ACC_PAD = 10240


def _sc_mesh():
    return plsc.VectorSubcoreMesh(core_axis_name="c", subcore_axis_name="s")


def _gather_body(table_hbm, idx_hbm, out_hbm, idx_v, rows_v, sem):
    c = lax.axis_index("c")
    s = lax.axis_index("s")
    w = s * 2 + c
    nchunks = idx_hbm.shape[0] // _CHUNK
    base_n = nchunks // _NW
    nk = base_n + jnp.where(w < (nchunks - base_n * _NW), 1, 0)

    def load(k):
        b = lax.rem(k, 2)
        base = (w + _NW * k) * _CHUNK
        pltpu.sync_copy(idx_hbm.at[pl.ds(base, _CHUNK)], idx_v.at[b])
        pltpu.async_copy(table_hbm.at[idx_v.at[b]], rows_v.at[b], sem.at[b])

    load(0)

    def body(k, carry):
        b = lax.rem(k, 2)

        @pl.when(k + 1 < nk)
        def _():
            load(k + 1)

        pltpu.make_async_copy(table_hbm.at[idx_v.at[b]], rows_v.at[b],
                              sem.at[b]).wait()
        base = (w + _NW * k) * _CHUNK
        pltpu.sync_copy(rows_v.at[b], out_hbm.at[pl.ds(base, _CHUNK)])
        return carry

    lax.fori_loop(0, nk, body, 0)


def _sc_gather(table, idx):
    e = idx.shape[0]
    dh = table.shape[1]
    k = pl.kernel(
        _gather_body,
        out_type=jax.ShapeDtypeStruct((e, dh), jnp.float32),
        mesh=_sc_mesh(),
        scratch_types=[
            pltpu.VMEM((2, _CHUNK), jnp.int32),
            pltpu.VMEM((2, _CHUNK, dh), jnp.float32),
            pltpu.SemaphoreType.DMA((2,)),
        ],
        compiler_params=pltpu.CompilerParams(use_tc_tiling_on_sc=False),
    )
    return k(table, idx)


def _scatter_body(with_deg, msg_hbm, idx_hbm, *refs):
    if with_deg:
        (out_hbm, deg_hbm, acc_sh, deg_sh, idx_v, rows_v, fill_v, sem) = refs
    else:
        (out_hbm, acc_sh, idx_v, rows_v, fill_v, sem) = refs
        deg_hbm = deg_sh = None
    c = lax.axis_index("c")
    s = lax.axis_index("s")
    dh = rows_v.shape[2]
    stripe = _ACC_PAD // 16

    def zb(i, carry):
        fill_v[i, :] = jnp.zeros((16,), jnp.float32)
        return carry

    lax.fori_loop(0, _CHUNK, zb, 0)
    for j in range(stripe // _CHUNK):
        off = s * stripe + j * _CHUNK
        pltpu.sync_copy(fill_v, acc_sh.at[pl.ds(off, _CHUNK)])
        if with_deg:
            pltpu.sync_copy(fill_v, deg_sh.at[pl.ds(off, _CHUNK)])
    if with_deg:
        def ob(i, carry):
            fill_v[i, :] = jnp.ones((16,), jnp.float32)
            return carry

        lax.fori_loop(0, _CHUNK, ob, 0)
    plsc.subcore_barrier()

    nchunks = idx_hbm.shape[0] // _CHUNK
    half = nchunks // 2
    base_n = half // 16
    nk = base_n + jnp.where(s < (half - base_n * 16), 1, 0)

    def load(k):
        b = lax.rem(k, 2)
        base = (c * half + s + 16 * k) * _CHUNK
        pltpu.sync_copy(idx_hbm.at[pl.ds(base, _CHUNK)], idx_v.at[b])
        pltpu.async_copy(msg_hbm.at[pl.ds(base, _CHUNK)], rows_v.at[b],
                         sem.at[b])

    load(0)

    def body(k, carry):
        b = lax.rem(k, 2)

        @pl.when(k + 1 < nk)
        def _():
            load(k + 1)

        base = (c * half + s + 16 * k) * _CHUNK
        pltpu.make_async_copy(msg_hbm.at[pl.ds(base, _CHUNK)], rows_v.at[b],
                              sem.at[b]).wait()
        pltpu.sync_copy(rows_v.at[b], acc_sh.at[idx_v.at[b]], add=True)
        if with_deg:
            pltpu.sync_copy(fill_v, deg_sh.at[idx_v.at[b]], add=True)
        return carry

    lax.fori_loop(0, nk, body, 0)
    plsc.subcore_barrier()

    n = out_hbm.shape[1]
    lo = s * stripe
    full = n // stripe
    tail = n - full * stripe

    @pl.when(s < full)
    def _():
        pltpu.sync_copy(acc_sh.at[pl.ds(lo, stripe)], out_hbm.at[c, pl.ds(lo, stripe)])
        if with_deg:
            pltpu.sync_copy(deg_sh.at[pl.ds(lo, stripe)], deg_hbm.at[c, pl.ds(lo, stripe)])

    if tail:
        @pl.when(s == full)
        def _():
            pltpu.sync_copy(acc_sh.at[pl.ds(full * stripe, tail)],
                            out_hbm.at[c, pl.ds(full * stripe, tail)])
            if with_deg:
                pltpu.sync_copy(deg_sh.at[pl.ds(full * stripe, tail)],
                                deg_hbm.at[c, pl.ds(full * stripe, tail)])


def _sc_scatter(msg, idx, n, with_deg):
    dh = msg.shape[1]
    out_t = jax.ShapeDtypeStruct((2, n, dh), jnp.float32)
    scratch = [pltpu.VMEM_SHARED((_ACC_PAD, dh), jnp.float32)]
    if with_deg:
        scratch.append(pltpu.VMEM_SHARED((_ACC_PAD, dh), jnp.float32))
    scratch += [
        pltpu.VMEM((2, _CHUNK), jnp.int32),
        pltpu.VMEM((2, _CHUNK, dh), jnp.float32),
        pltpu.VMEM((_CHUNK, dh), jnp.float32),
        pltpu.SemaphoreType.DMA((2,)),
    ]
    k = pl.kernel(
        functools.partial(_scatter_body, with_deg),
        out_type=(out_t, out_t) if with_deg else out_t,
        mesh=_sc_mesh(),
        scratch_types=scratch,
        compiler_params=pltpu.CompilerParams(use_tc_tiling_on_sc=False),
    )
    return k(msg, idx)


def _lin0_body(x_ref, w_ref, b_ref, o_ref):
    o_ref[...] = jax.nn.relu(
        jnp.dot(x_ref[...], w_ref[...], preferred_element_type=jnp.float32,
                precision=lax.Precision.HIGHEST) + b_ref[...])


def _lin0(x, w, b):
    n, din = x.shape
    dh = w.shape[1]
    blk = 1000
    return pl.pallas_call(
        _lin0_body,
        grid=(n // blk,),
        in_specs=[
            pl.BlockSpec((blk, din), lambda i: (i, 0)),
            pl.BlockSpec((din, dh), lambda i: (0, 0)),
            pl.BlockSpec((1, dh), lambda i: (0, 0)),
        ],
        out_specs=pl.BlockSpec((blk, dh), lambda i: (i, 0)),
        out_shape=jax.ShapeDtypeStruct((n, dh), jnp.float32),
    )(x, w, b)


def _edge_body(ea_ref, os_ref, w1_ref, b1_ref, w2_ref, b2_ref, q_ref, r_ref,
               msg_ref):
    bf = jnp.bfloat16
    eh = jax.nn.relu(
        jnp.dot(ea_ref[...].astype(bf), w1_ref[...].astype(bf),
                preferred_element_type=jnp.float32) + b1_ref[...])
    we = jnp.dot(eh.astype(bf), w2_ref[...].astype(bf),
                 preferred_element_type=jnp.float32) + b2_ref[...]
    rep = jnp.dot(os_ref[...].astype(bf), q_ref[...],
                  preferred_element_type=jnp.float32)
    msg_ref[...] = jnp.dot((we * rep).astype(bf), r_ref[...],
                           preferred_element_type=jnp.float32)


def _edge_msgs(ea, out_src, w1, b1, w2, b2, qm, rm):
    e, de = ea.shape
    dh = out_src.shape[1]
    eh = w1.shape[1]
    d2 = w2.shape[1]
    blk = 3200
    return pl.pallas_call(
        _edge_body,
        grid=(e // blk,),
        in_specs=[
            pl.BlockSpec((blk, de), lambda i: (i, 0)),
            pl.BlockSpec((blk, dh), lambda i: (i, 0)),
            pl.BlockSpec((de, eh), lambda i: (0, 0)),
            pl.BlockSpec((1, eh), lambda i: (0, 0)),
            pl.BlockSpec((eh, d2), lambda i: (0, 0)),
            pl.BlockSpec((1, d2), lambda i: (0, 0)),
            pl.BlockSpec((dh, d2), lambda i: (0, 0)),
            pl.BlockSpec((d2, dh), lambda i: (0, 0)),
        ],
        out_specs=pl.BlockSpec((blk, dh), lambda i: (i, 0)),
        out_shape=jax.ShapeDtypeStruct((e, dh), jnp.float32),
    )(ea, out_src, w1, b1, w2, b2, qm, rm)


def _gru_body(p_ref, dp_ref, cb_ref, h_ref, wih_ref, whh_ref, bih_ref,
              bhh_ref, o_ref):
    hi = lax.Precision.HIGHEST
    dh = h_ref.shape[1]
    agg = p_ref[0] + p_ref[1]
    deg = jnp.maximum(dp_ref[0] + dp_ref[1], 1.0)
    m = jax.nn.relu(agg / deg + cb_ref[...])
    gi = jnp.dot(m, wih_ref[...], preferred_element_type=jnp.float32,
                 precision=hi) + bih_ref[...]
    h = h_ref[...]
    gh = jnp.dot(h, whh_ref[...], preferred_element_type=jnp.float32,
                 precision=hi) + bhh_ref[...]
    r = jax.nn.sigmoid(gi[:, :dh] + gh[:, :dh])
    z = jax.nn.sigmoid(gi[:, dh:2 * dh] + gh[:, dh:2 * dh])
    nn = jnp.tanh(gi[:, 2 * dh:] + r * gh[:, 2 * dh:])
    o_ref[...] = (1.0 - z) * nn + z * h


def _gru(partials, degp, cb, h, wih, whh, bih, bhh):
    n, dh = h.shape
    d3 = wih.shape[1]
    blk = 2000
    return pl.pallas_call(
        _gru_body,
        grid=(n // blk,),
        in_specs=[
            pl.BlockSpec((2, blk, dh), lambda i: (0, i, 0)),
            pl.BlockSpec((2, blk, dh), lambda i: (0, i, 0)),
            pl.BlockSpec((1, dh), lambda i: (0, 0)),
            pl.BlockSpec((blk, dh), lambda i: (i, 0)),
            pl.BlockSpec((dh, d3), lambda i: (0, 0)),
            pl.BlockSpec((dh, d3), lambda i: (0, 0)),
            pl.BlockSpec((1, d3), lambda i: (0, 0)),
            pl.BlockSpec((1, d3), lambda i: (0, 0)),
        ],
        out_specs=pl.BlockSpec((blk, dh), lambda i: (i, 0)),
        out_shape=jax.ShapeDtypeStruct((n, dh), jnp.float32),
    )(partials, degp, cb, h, wih, whh, bih, bhh)


def _s2s_body(out_ref, batch_ref, wih_ref, whh_ref, bih_ref, bhh_ref,
              l1w_ref, l1b_ref, l2w_ref, l2b_ref, y_ref):
    hi = lax.Precision.HIGHEST
    xo = out_ref[...]
    n, dh = xo.shape
    rows = lax.broadcasted_iota(jnp.int32, (_B, n), 0)
    mb = jnp.broadcast_to(batch_ref[...], (_B, n))
    mask = rows == mb

    q_star = jnp.zeros((_B, 2 * dh), jnp.float32)
    hh = jnp.zeros((_B, dh), jnp.float32)
    cc = jnp.zeros((_B, dh), jnp.float32)
    for _ in range(3):
        g = (jnp.dot(q_star, wih_ref[...], preferred_element_type=jnp.float32,
                     precision=hi) + bih_ref[...] +
             jnp.dot(hh, whh_ref[...], preferred_element_type=jnp.float32,
                     precision=hi) + bhh_ref[...])
        ig = jax.nn.sigmoid(g[:, :dh])
        fg = jax.nn.sigmoid(g[:, dh:2 * dh])
        cg = jnp.tanh(g[:, 2 * dh:3 * dh])
        og = jax.nn.sigmoid(g[:, 3 * dh:])
        cc = fg * cc + ig * cg
        hh = og * jnp.tanh(cc)
        e_all = lax.dot_general(hh, xo, (((1,), (1,)), ((), ())),
                                preferred_element_type=jnp.float32,
                                precision=hi)
        e_masked = jnp.where(mask, e_all, -jnp.inf)
        emax = jnp.max(e_masked, axis=1, keepdims=True)
        emax = jnp.where(jnp.isfinite(emax), emax, 0.0)
        a = jnp.where(mask, jnp.exp(e_all - emax), 0.0)
        denom = jnp.sum(a, axis=1, keepdims=True)
        a = a / jnp.maximum(denom, 1e-30)
        r_read = jnp.dot(a, xo, preferred_element_type=jnp.float32,
                         precision=hi)
        q_star = jnp.concatenate([hh, r_read], axis=1)
    y = jax.nn.relu(
        jnp.dot(q_star, l1w_ref[...], preferred_element_type=jnp.float32,
                precision=hi) + l1b_ref[...])
    y_ref[...] = jnp.dot(y, l2w_ref[...], preferred_element_type=jnp.float32,
                         precision=hi) + l2b_ref[...]


def _s2s(out, batch2d, wih, whh, bih, bhh, l1w, l1b, l2w, l2b):
    dout = l2w.shape[1]
    return pl.pallas_call(
        _s2s_body,
        out_shape=jax.ShapeDtypeStruct((_B, dout), jnp.float32),
    )(out, batch2d, wih, whh, bih, bhh, l1w, l1b, l2w, l2b)


def kernel(x, edge_index, edge_attr, batch, lin0_W, lin0_b, edge_W1, edge_b1,
           edge_W2, edge_b2, conv_b, gru_Wih, gru_Whh, gru_bih, gru_bhh,
           lstm_Wih, lstm_Whh, lstm_bih, lstm_bhh, lin1_W, lin1_b, lin2_W,
           lin2_b):
    n = x.shape[0]
    dh = lin0_W.shape[1]
    d2 = dh * dh
    nlayers = edge_W1.shape[0]
    src = edge_index[0]
    dst = edge_index[1]

    j = jnp.arange(d2, dtype=jnp.int32)
    qm = (j[None, :] // dh == jnp.arange(dh, dtype=jnp.int32)[:, None]
          ).astype(jnp.bfloat16)
    rm = (j[:, None] % dh == jnp.arange(dh, dtype=jnp.int32)[None, :]
          ).astype(jnp.bfloat16)

    h = _lin0(x, lin0_W, lin0_b.reshape(1, -1))
    degp = None
    for li in range(nlayers):
        out_src = _sc_gather(h, src)
        msg = _edge_msgs(edge_attr, out_src, edge_W1[li],
                         edge_b1[li].reshape(1, -1), edge_W2[li],
                         edge_b2[li].reshape(1, -1), qm, rm)
        if degp is None:
            partials, degp = _sc_scatter(msg, dst, n, True)
        else:
            partials = _sc_scatter(msg, dst, n, False)
        h = _gru(partials, degp, conv_b[li].reshape(1, -1), h, gru_Wih,
                 gru_Whh, gru_bih.reshape(1, -1), gru_bhh.reshape(1, -1))
    return _s2s(h, batch.reshape(1, -1), lstm_Wih, lstm_Whh,
                lstm_bih.reshape(1, -1), lstm_bhh.reshape(1, -1), lin1_W,
                lin1_b.reshape(1, -1), lin2_W, lin2_b.reshape(1, -1))


# speedup vs baseline: 4.1276x; 1.0365x over previous
import functools

import jax
import jax.numpy as jnp
from jax import lax
from jax.experimental import pallas as pl
from jax.experimental.pallas import tpu as pltpu
from jax.experimental.pallas import tpu_sc as plsc

_B = 64

_CHUNK = 128
_NW = 32
_ACC_PAD = 10240


def _sc_mesh():
    return plsc.VectorSubcoreMesh(core_axis_name="c", subcore_axis_name="s")


def _gather_body(table_hbm, idx_hbm, out_hbm, idx_v, rows_v, sem):
    c = lax.axis_index("c")
    s = lax.axis_index("s")
    w = s * 2 + c
    nchunks = idx_hbm.shape[0] // _CHUNK
    base_n = nchunks // _NW
    nk = base_n + jnp.where(w < (nchunks - base_n * _NW), 1, 0)

    def load(k):
        b = lax.rem(k, 2)
        base = (w + _NW * k) * _CHUNK
        pltpu.sync_copy(idx_hbm.at[pl.ds(base, _CHUNK)], idx_v.at[b])
        pltpu.async_copy(table_hbm.at[idx_v.at[b]], rows_v.at[b], sem.at[b])

    load(0)

    def body(k, carry):
        b = lax.rem(k, 2)

        @pl.when(k + 1 < nk)
        def _():
            load(k + 1)

        pltpu.make_async_copy(table_hbm.at[idx_v.at[b]], rows_v.at[b],
                              sem.at[b]).wait()
        base = (w + _NW * k) * _CHUNK
        pltpu.sync_copy(rows_v.at[b], out_hbm.at[pl.ds(base, _CHUNK)])
        return carry

    lax.fori_loop(0, nk, body, 0)


def _sc_gather(table, idx):
    e = idx.shape[0]
    dh = table.shape[1]
    k = pl.kernel(
        _gather_body,
        out_type=jax.ShapeDtypeStruct((e, dh), jnp.float32),
        mesh=_sc_mesh(),
        scratch_types=[
            pltpu.VMEM((2, _CHUNK), jnp.int32),
            pltpu.VMEM((2, _CHUNK, dh), jnp.float32),
            pltpu.SemaphoreType.DMA((2,)),
        ],
        compiler_params=pltpu.CompilerParams(use_tc_tiling_on_sc=False),
    )
    return k(table, idx)


def _scatter_body(with_deg, msg_hbm, idx_hbm, *refs):
    if with_deg:
        (out_hbm, deg_hbm, acc_sh, deg_sh, idx_v, rows_v, fill_v, sem) = refs
    else:
        (out_hbm, acc_sh, idx_v, rows_v, fill_v, sem) = refs
        deg_hbm = deg_sh = None
    c = lax.axis_index("c")
    s = lax.axis_index("s")
    dh = rows_v.shape[2]
    stripe = _ACC_PAD // 16

    def zb(i, carry):
        fill_v[i, :] = jnp.zeros((16,), jnp.float32)
        return carry

    lax.fori_loop(0, _CHUNK, zb, 0)
    for j in range(stripe // _CHUNK):
        off = s * stripe + j * _CHUNK
        pltpu.sync_copy(fill_v, acc_sh.at[pl.ds(off, _CHUNK)])
        if with_deg:
            pltpu.sync_copy(fill_v, deg_sh.at[pl.ds(off, _CHUNK)])
    if with_deg:
        def ob(i, carry):
            fill_v[i, :] = jnp.ones((16,), jnp.float32)
            return carry

        lax.fori_loop(0, _CHUNK, ob, 0)
    plsc.subcore_barrier()

    nchunks = idx_hbm.shape[0] // _CHUNK
    half = nchunks // 2
    base_n = half // 16
    nk = base_n + jnp.where(s < (half - base_n * 16), 1, 0)

    def load(k):
        b = lax.rem(k, 2)
        base = (c * half + s + 16 * k) * _CHUNK
        pltpu.sync_copy(idx_hbm.at[pl.ds(base, _CHUNK)], idx_v.at[b])
        pltpu.async_copy(msg_hbm.at[pl.ds(base, _CHUNK)], rows_v.at[b],
                         sem.at[b])

    load(0)

    def body(k, carry):
        b = lax.rem(k, 2)

        @pl.when(k + 1 < nk)
        def _():
            load(k + 1)

        base = (c * half + s + 16 * k) * _CHUNK
        pltpu.make_async_copy(msg_hbm.at[pl.ds(base, _CHUNK)], rows_v.at[b],
                              sem.at[b]).wait()
        pltpu.sync_copy(rows_v.at[b], acc_sh.at[idx_v.at[b]], add=True)
        if with_deg:
            pltpu.sync_copy(fill_v, deg_sh.at[idx_v.at[b]], add=True)
        return carry

    lax.fori_loop(0, nk, body, 0)
    plsc.subcore_barrier()

    n = out_hbm.shape[1]
    lo = s * stripe
    full = n // stripe
    tail = n - full * stripe

    @pl.when(s < full)
    def _():
        pltpu.sync_copy(acc_sh.at[pl.ds(lo, stripe)], out_hbm.at[c, pl.ds(lo, stripe)])
        if with_deg:
            pltpu.sync_copy(deg_sh.at[pl.ds(lo, stripe)], deg_hbm.at[c, pl.ds(lo, stripe)])

    if tail:
        @pl.when(s == full)
        def _():
            pltpu.sync_copy(acc_sh.at[pl.ds(full * stripe, tail)],
                            out_hbm.at[c, pl.ds(full * stripe, tail)])
            if with_deg:
                pltpu.sync_copy(deg_sh.at[pl.ds(full * stripe, tail)],
                                deg_hbm.at[c, pl.ds(full * stripe, tail)])


def _sc_scatter(msg, idx, n, with_deg):
    dh = msg.shape[1]
    out_t = jax.ShapeDtypeStruct((2, n, dh), jnp.float32)
    scratch = [pltpu.VMEM_SHARED((_ACC_PAD, dh), jnp.float32)]
    if with_deg:
        scratch.append(pltpu.VMEM_SHARED((_ACC_PAD, dh), jnp.float32))
    scratch += [
        pltpu.VMEM((2, _CHUNK), jnp.int32),
        pltpu.VMEM((2, _CHUNK, dh), jnp.float32),
        pltpu.VMEM((_CHUNK, dh), jnp.float32),
        pltpu.SemaphoreType.DMA((2,)),
    ]
    k = pl.kernel(
        functools.partial(_scatter_body, with_deg),
        out_type=(out_t, out_t) if with_deg else out_t,
        mesh=_sc_mesh(),
        scratch_types=scratch,
        compiler_params=pltpu.CompilerParams(use_tc_tiling_on_sc=False),
    )
    return k(msg, idx)


def _lin0_body(x_ref, w_ref, b_ref, o_ref):
    o_ref[...] = jax.nn.relu(
        jnp.dot(x_ref[...], w_ref[...], preferred_element_type=jnp.float32,
                precision=lax.Precision.HIGHEST) + b_ref[...])


def _lin0(x, w, b):
    n, din = x.shape
    dh = w.shape[1]
    blk = 1000
    return pl.pallas_call(
        _lin0_body,
        grid=(n // blk,),
        in_specs=[
            pl.BlockSpec((blk, din), lambda i: (i, 0)),
            pl.BlockSpec((din, dh), lambda i: (0, 0)),
            pl.BlockSpec((1, dh), lambda i: (0, 0)),
        ],
        out_specs=pl.BlockSpec((blk, dh), lambda i: (i, 0)),
        out_shape=jax.ShapeDtypeStruct((n, dh), jnp.float32),
    )(x, w, b)


def _edge_body(ea_ref, os_ref, w1_ref, b1_ref, w2_ref, b2_ref, q_ref, r_ref,
               msg_ref):
    bf = jnp.bfloat16
    eh = jax.nn.relu(
        jnp.dot(ea_ref[...].astype(bf), w1_ref[...].astype(bf),
                preferred_element_type=jnp.float32) + b1_ref[...])
    we = jnp.dot(eh.astype(bf), w2_ref[...].astype(bf),
                 preferred_element_type=jnp.float32) + b2_ref[...]
    rep = jnp.dot(os_ref[...].astype(bf), q_ref[...],
                  preferred_element_type=jnp.float32)
    msg_ref[...] = jnp.dot((we * rep).astype(bf), r_ref[...],
                           preferred_element_type=jnp.float32)


def _edge_msgs(ea, out_src, w1, b1, w2, b2, qm, rm):
    e, de = ea.shape
    dh = out_src.shape[1]
    eh = w1.shape[1]
    d2 = w2.shape[1]
    blk = 6400
    return pl.pallas_call(
        _edge_body,
        grid=(e // blk,),
        in_specs=[
            pl.BlockSpec((blk, de), lambda i: (i, 0)),
            pl.BlockSpec((blk, dh), lambda i: (i, 0)),
            pl.BlockSpec((de, eh), lambda i: (0, 0)),
            pl.BlockSpec((1, eh), lambda i: (0, 0)),
            pl.BlockSpec((eh, d2), lambda i: (0, 0)),
            pl.BlockSpec((1, d2), lambda i: (0, 0)),
            pl.BlockSpec((dh, d2), lambda i: (0, 0)),
            pl.BlockSpec((d2, dh), lambda i: (0, 0)),
        ],
        out_specs=pl.BlockSpec((blk, dh), lambda i: (i, 0)),
        out_shape=jax.ShapeDtypeStruct((e, dh), jnp.float32),
    )(ea, out_src, w1, b1, w2, b2, qm, rm)


def _gru_body(p_ref, dp_ref, cb_ref, h_ref, wih_ref, whh_ref, bih_ref,
              bhh_ref, o_ref):
    hi = lax.Precision.HIGHEST
    dh = h_ref.shape[1]
    agg = p_ref[0] + p_ref[1]
    deg = jnp.maximum(dp_ref[0] + dp_ref[1], 1.0)
    m = jax.nn.relu(agg / deg + cb_ref[...])
    gi = jnp.dot(m, wih_ref[...], preferred_element_type=jnp.float32,
                 precision=hi) + bih_ref[...]
    h = h_ref[...]
    gh = jnp.dot(h, whh_ref[...], preferred_element_type=jnp.float32,
                 precision=hi) + bhh_ref[...]
    r = jax.nn.sigmoid(gi[:, :dh] + gh[:, :dh])
    z = jax.nn.sigmoid(gi[:, dh:2 * dh] + gh[:, dh:2 * dh])
    nn = jnp.tanh(gi[:, 2 * dh:] + r * gh[:, 2 * dh:])
    o_ref[...] = (1.0 - z) * nn + z * h


def _gru(partials, degp, cb, h, wih, whh, bih, bhh):
    n, dh = h.shape
    d3 = wih.shape[1]
    blk = 2000
    return pl.pallas_call(
        _gru_body,
        grid=(n // blk,),
        in_specs=[
            pl.BlockSpec((2, blk, dh), lambda i: (0, i, 0)),
            pl.BlockSpec((2, blk, dh), lambda i: (0, i, 0)),
            pl.BlockSpec((1, dh), lambda i: (0, 0)),
            pl.BlockSpec((blk, dh), lambda i: (i, 0)),
            pl.BlockSpec((dh, d3), lambda i: (0, 0)),
            pl.BlockSpec((dh, d3), lambda i: (0, 0)),
            pl.BlockSpec((1, d3), lambda i: (0, 0)),
            pl.BlockSpec((1, d3), lambda i: (0, 0)),
        ],
        out_specs=pl.BlockSpec((blk, dh), lambda i: (i, 0)),
        out_shape=jax.ShapeDtypeStruct((n, dh), jnp.float32),
    )(partials, degp, cb, h, wih, whh, bih, bhh)


def _s2s_body(out_ref, batch_ref, wih_ref, whh_ref, bih_ref, bhh_ref,
              l1w_ref, l1b_ref, l2w_ref, l2b_ref, y_ref):
    hi = lax.Precision.HIGHEST
    xo = out_ref[...]
    n, dh = xo.shape
    rows = lax.broadcasted_iota(jnp.int32, (_B, n), 0)
    mb = jnp.broadcast_to(batch_ref[...], (_B, n))
    mask = rows == mb

    q_star = jnp.zeros((_B, 2 * dh), jnp.float32)
    hh = jnp.zeros((_B, dh), jnp.float32)
    cc = jnp.zeros((_B, dh), jnp.float32)
    for _ in range(3):
        g = (jnp.dot(q_star, wih_ref[...], preferred_element_type=jnp.float32,
                     precision=hi) + bih_ref[...] +
             jnp.dot(hh, whh_ref[...], preferred_element_type=jnp.float32,
                     precision=hi) + bhh_ref[...])
        ig = jax.nn.sigmoid(g[:, :dh])
        fg = jax.nn.sigmoid(g[:, dh:2 * dh])
        cg = jnp.tanh(g[:, 2 * dh:3 * dh])
        og = jax.nn.sigmoid(g[:, 3 * dh:])
        cc = fg * cc + ig * cg
        hh = og * jnp.tanh(cc)
        e_all = lax.dot_general(hh, xo, (((1,), (1,)), ((), ())),
                                preferred_element_type=jnp.float32,
                                precision=hi)
        e_masked = jnp.where(mask, e_all, -jnp.inf)
        emax = jnp.max(e_masked, axis=1, keepdims=True)
        emax = jnp.where(jnp.isfinite(emax), emax, 0.0)
        a = jnp.where(mask, jnp.exp(e_all - emax), 0.0)
        denom = jnp.sum(a, axis=1, keepdims=True)
        a = a / jnp.maximum(denom, 1e-30)
        r_read = jnp.dot(a, xo, preferred_element_type=jnp.float32,
                         precision=hi)
        q_star = jnp.concatenate([hh, r_read], axis=1)
    y = jax.nn.relu(
        jnp.dot(q_star, l1w_ref[...], preferred_element_type=jnp.float32,
                precision=hi) + l1b_ref[...])
    y_ref[...] = jnp.dot(y, l2w_ref[...], preferred_element_type=jnp.float32,
                         precision=hi) + l2b_ref[...]


def _s2s(out, batch2d, wih, whh, bih, bhh, l1w, l1b, l2w, l2b):
    dout = l2w.shape[1]
    return pl.pallas_call(
        _s2s_body,
        out_shape=jax.ShapeDtypeStruct((_B, dout), jnp.float32),
    )(out, batch2d, wih, whh, bih, bhh, l1w, l1b, l2w, l2b)


def kernel(x, edge_index, edge_attr, batch, lin0_W, lin0_b, edge_W1, edge_b1,
           edge_W2, edge_b2, conv_b, gru_Wih, gru_Whh, gru_bih, gru_bhh,
           lstm_Wih, lstm_Whh, lstm_bih, lstm_bhh, lin1_W, lin1_b, lin2_W,
           lin2_b):
    n = x.shape[0]
    dh = lin0_W.shape[1]
    d2 = dh * dh
    nlayers = edge_W1.shape[0]
    src = edge_index[0]
    dst = edge_index[1]

    j = jnp.arange(d2, dtype=jnp.int32)
    qm = (j[None, :] // dh == jnp.arange(dh, dtype=jnp.int32)[:, None]
          ).astype(jnp.bfloat16)
    rm = (j[:, None] % dh == jnp.arange(dh, dtype=jnp.int32)[None, :]
          ).astype(jnp.bfloat16)

    h = _lin0(x, lin0_W, lin0_b.reshape(1, -1))
    degp = None
    for li in range(nlayers):
        out_src = _sc_gather(h, src)
        msg = _edge_msgs(edge_attr, out_src, edge_W1[li],
                         edge_b1[li].reshape(1, -1), edge_W2[li],
                         edge_b2[li].reshape(1, -1), qm, rm)
        if degp is None:
            partials, degp = _sc_scatter(msg, dst, n, True)
        else:
            partials = _sc_scatter(msg, dst, n, False)
        h = _gru(partials, degp, conv_b[li].reshape(1, -1), h, gru_Wih,
                 gru_Whh, gru_bih.reshape(1, -1), gru_bhh.reshape(1, -1))
    return _s2s(h, batch.reshape(1, -1), lstm_Wih, lstm_Whh,
                lstm_bih.reshape(1, -1), lstm_bhh.reshape(1, -1), lin1_W,
                lin1_b.reshape(1, -1), lin2_W, lin2_b.reshape(1, -1))


# speedup vs baseline: 4.4458x; 1.0771x over previous
import functools

import jax
import jax.numpy as jnp
from jax import lax
from jax.experimental import pallas as pl
from jax.experimental.pallas import tpu as pltpu
from jax.experimental.pallas import tpu_sc as plsc

_B = 64

_CHUNK = 128
_NW = 32
_ACC_PAD = 10240


def _sc_mesh():
    return plsc.VectorSubcoreMesh(core_axis_name="c", subcore_axis_name="s")


def _gather_body(table_hbm, idx_hbm, out_hbm, idx_v, rows_v, semA, semB,
                 semC):
    c = lax.axis_index("c")
    s = lax.axis_index("s")
    w = s * 2 + c
    nchunks = idx_hbm.shape[0] // _CHUNK
    base_n = nchunks // _NW
    nk = base_n + jnp.where(w < (nchunks - base_n * _NW), 1, 0)

    def idx_cp(k):
        b = lax.rem(k, 4)
        base = (w + _NW * k) * _CHUNK
        return pltpu.make_async_copy(idx_hbm.at[pl.ds(base, _CHUNK)],
                                     idx_v.at[b], semA.at[b])

    def gat_cp(k):
        b = lax.rem(k, 4)
        return pltpu.make_async_copy(table_hbm.at[idx_v.at[b]],
                                     rows_v.at[b], semB.at[b])

    def out_cp(k):
        b = lax.rem(k, 4)
        base = (w + _NW * k) * _CHUNK
        return pltpu.make_async_copy(rows_v.at[b],
                                     out_hbm.at[pl.ds(base, _CHUNK)],
                                     semC.at[b])

    idx_cp(0).start()
    idx_cp(1).start()
    idx_cp(0).wait()
    gat_cp(0).start()

    def body(k, carry):
        @pl.when(k >= 2)
        def _():
            out_cp(k - 2).wait()

        @pl.when(k + 2 < nk)
        def _():
            idx_cp(k + 2).start()

        @pl.when(k + 1 < nk)
        def _():
            idx_cp(k + 1).wait()
            gat_cp(k + 1).start()

        gat_cp(k).wait()
        out_cp(k).start()
        return carry

    lax.fori_loop(0, nk, body, 0)

    @pl.when(nk >= 2)
    def _():
        out_cp(nk - 2).wait()

    out_cp(nk - 1).wait()


def _sc_gather(table, idx):
    e = idx.shape[0]
    dh = table.shape[1]
    k = pl.kernel(
        _gather_body,
        out_type=jax.ShapeDtypeStruct((e, dh), jnp.float32),
        mesh=_sc_mesh(),
        scratch_types=[
            pltpu.VMEM((4, _CHUNK), jnp.int32),
            pltpu.VMEM((4, _CHUNK, dh), jnp.float32),
            pltpu.SemaphoreType.DMA((4,)),
            pltpu.SemaphoreType.DMA((4,)),
            pltpu.SemaphoreType.DMA((4,)),
        ],
        compiler_params=pltpu.CompilerParams(use_tc_tiling_on_sc=False),
    )
    return k(table, idx)


def _scatter_body(with_deg, msg_hbm, idx_hbm, *refs):
    if with_deg:
        (out_hbm, deg_hbm, acc_sh, deg_sh, idx_v, rows_v, fill_v, semA, semB,
         semC, semD) = refs
    else:
        (out_hbm, acc_sh, idx_v, rows_v, fill_v, semA, semB, semC) = refs
        deg_hbm = deg_sh = semD = None
    c = lax.axis_index("c")
    s = lax.axis_index("s")
    dh = rows_v.shape[2]
    stripe = _ACC_PAD // 16

    def zb(i, carry):
        fill_v[i, :] = jnp.zeros((16,), jnp.float32)
        return carry

    lax.fori_loop(0, _CHUNK, zb, 0)
    for j in range(stripe // _CHUNK):
        off = s * stripe + j * _CHUNK
        pltpu.sync_copy(fill_v, acc_sh.at[pl.ds(off, _CHUNK)])
        if with_deg:
            pltpu.sync_copy(fill_v, deg_sh.at[pl.ds(off, _CHUNK)])
    if with_deg:
        def ob(i, carry):
            fill_v[i, :] = jnp.ones((16,), jnp.float32)
            return carry

        lax.fori_loop(0, _CHUNK, ob, 0)
    plsc.subcore_barrier()

    nchunks = idx_hbm.shape[0] // _CHUNK
    half = nchunks // 2
    base_n = half // 16
    nk = base_n + jnp.where(s < (half - base_n * 16), 1, 0)

    def idx_cp(k):
        b = lax.rem(k, 4)
        base = (c * half + s + 16 * k) * _CHUNK
        return pltpu.make_async_copy(idx_hbm.at[pl.ds(base, _CHUNK)],
                                     idx_v.at[b], semA.at[b])

    def row_cp(k):
        b = lax.rem(k, 4)
        base = (c * half + s + 16 * k) * _CHUNK
        return pltpu.make_async_copy(msg_hbm.at[pl.ds(base, _CHUNK)],
                                     rows_v.at[b], semB.at[b])

    def start_add(k):
        b = lax.rem(k, 4)
        pltpu.async_copy(rows_v.at[b], acc_sh.at[idx_v.at[b]], semC.at[b],
                         add=True)
        if with_deg:
            pltpu.async_copy(fill_v, deg_sh.at[idx_v.at[b]], semD.at[b],
                             add=True)

    def wait_add(k):
        b = lax.rem(k, 4)
        pltpu.make_async_copy(rows_v.at[b], acc_sh.at[idx_v.at[b]],
                              semC.at[b]).wait()
        if with_deg:
            pltpu.make_async_copy(fill_v, deg_sh.at[idx_v.at[b]],
                                  semD.at[b]).wait()

    idx_cp(0).start()
    row_cp(0).start()
    idx_cp(1).start()
    row_cp(1).start()

    def body(k, carry):
        @pl.when(k >= 2)
        def _():
            wait_add(k - 2)

        @pl.when(k + 2 < nk)
        def _():
            idx_cp(k + 2).start()
            row_cp(k + 2).start()

        idx_cp(k).wait()
        row_cp(k).wait()
        start_add(k)
        return carry

    lax.fori_loop(0, nk, body, 0)

    @pl.when(nk >= 2)
    def _():
        wait_add(nk - 2)

    wait_add(nk - 1)
    plsc.subcore_barrier()

    n = out_hbm.shape[1]
    lo = s * stripe
    full = n // stripe
    tail = n - full * stripe

    @pl.when(s < full)
    def _():
        pltpu.sync_copy(acc_sh.at[pl.ds(lo, stripe)], out_hbm.at[c, pl.ds(lo, stripe)])
        if with_deg:
            pltpu.sync_copy(deg_sh.at[pl.ds(lo, stripe)], deg_hbm.at[c, pl.ds(lo, stripe)])

    if tail:
        @pl.when(s == full)
        def _():
            pltpu.sync_copy(acc_sh.at[pl.ds(full * stripe, tail)],
                            out_hbm.at[c, pl.ds(full * stripe, tail)])
            if with_deg:
                pltpu.sync_copy(deg_sh.at[pl.ds(full * stripe, tail)],
                                deg_hbm.at[c, pl.ds(full * stripe, tail)])


def _sc_scatter(msg, idx, n, with_deg):
    dh = msg.shape[1]
    out_t = jax.ShapeDtypeStruct((2, n, dh), jnp.float32)
    scratch = [pltpu.VMEM_SHARED((_ACC_PAD, dh), jnp.float32)]
    if with_deg:
        scratch.append(pltpu.VMEM_SHARED((_ACC_PAD, dh), jnp.float32))
    scratch += [
        pltpu.VMEM((4, _CHUNK), jnp.int32),
        pltpu.VMEM((4, _CHUNK, dh), jnp.float32),
        pltpu.VMEM((_CHUNK, dh), jnp.float32),
        pltpu.SemaphoreType.DMA((4,)),
        pltpu.SemaphoreType.DMA((4,)),
        pltpu.SemaphoreType.DMA((4,)),
    ]
    if with_deg:
        scratch.append(pltpu.SemaphoreType.DMA((4,)))
    k = pl.kernel(
        functools.partial(_scatter_body, with_deg),
        out_type=(out_t, out_t) if with_deg else out_t,
        mesh=_sc_mesh(),
        scratch_types=scratch,
        compiler_params=pltpu.CompilerParams(use_tc_tiling_on_sc=False),
    )
    return k(msg, idx)


def _lin0_body(x_ref, w_ref, b_ref, o_ref):
    o_ref[...] = jax.nn.relu(
        jnp.dot(x_ref[...], w_ref[...], preferred_element_type=jnp.float32,
                precision=lax.Precision.HIGHEST) + b_ref[...])


def _lin0(x, w, b):
    n, din = x.shape
    dh = w.shape[1]
    blk = 1000
    return pl.pallas_call(
        _lin0_body,
        grid=(n // blk,),
        in_specs=[
            pl.BlockSpec((blk, din), lambda i: (i, 0)),
            pl.BlockSpec((din, dh), lambda i: (0, 0)),
            pl.BlockSpec((1, dh), lambda i: (0, 0)),
        ],
        out_specs=pl.BlockSpec((blk, dh), lambda i: (i, 0)),
        out_shape=jax.ShapeDtypeStruct((n, dh), jnp.float32),
    )(x, w, b)


def _edge_body(ea_ref, os_ref, w1_ref, b1_ref, w2_ref, b2_ref, q_ref, r_ref,
               msg_ref):
    bf = jnp.bfloat16
    eh = jax.nn.relu(
        jnp.dot(ea_ref[...].astype(bf), w1_ref[...].astype(bf),
                preferred_element_type=jnp.float32) + b1_ref[...])
    we = jnp.dot(eh.astype(bf), w2_ref[...].astype(bf),
                 preferred_element_type=jnp.float32) + b2_ref[...]
    rep = jnp.dot(os_ref[...].astype(bf), q_ref[...],
                  preferred_element_type=jnp.float32)
    msg_ref[...] = jnp.dot((we * rep).astype(bf), r_ref[...],
                           preferred_element_type=jnp.float32)


def _edge_msgs(ea, out_src, w1, b1, w2, b2, qm, rm):
    e, de = ea.shape
    dh = out_src.shape[1]
    eh = w1.shape[1]
    d2 = w2.shape[1]
    blk = 6400
    return pl.pallas_call(
        _edge_body,
        grid=(e // blk,),
        in_specs=[
            pl.BlockSpec((blk, de), lambda i: (i, 0)),
            pl.BlockSpec((blk, dh), lambda i: (i, 0)),
            pl.BlockSpec((de, eh), lambda i: (0, 0)),
            pl.BlockSpec((1, eh), lambda i: (0, 0)),
            pl.BlockSpec((eh, d2), lambda i: (0, 0)),
            pl.BlockSpec((1, d2), lambda i: (0, 0)),
            pl.BlockSpec((dh, d2), lambda i: (0, 0)),
            pl.BlockSpec((d2, dh), lambda i: (0, 0)),
        ],
        out_specs=pl.BlockSpec((blk, dh), lambda i: (i, 0)),
        out_shape=jax.ShapeDtypeStruct((e, dh), jnp.float32),
    )(ea, out_src, w1, b1, w2, b2, qm, rm)


def _gru_body(p_ref, dp_ref, cb_ref, h_ref, wih_ref, whh_ref, bih_ref,
              bhh_ref, o_ref):
    hi = lax.Precision.HIGHEST
    dh = h_ref.shape[1]
    agg = p_ref[0] + p_ref[1]
    deg = jnp.maximum(dp_ref[0] + dp_ref[1], 1.0)
    m = jax.nn.relu(agg / deg + cb_ref[...])
    gi = jnp.dot(m, wih_ref[...], preferred_element_type=jnp.float32,
                 precision=hi) + bih_ref[...]
    h = h_ref[...]
    gh = jnp.dot(h, whh_ref[...], preferred_element_type=jnp.float32,
                 precision=hi) + bhh_ref[...]
    r = jax.nn.sigmoid(gi[:, :dh] + gh[:, :dh])
    z = jax.nn.sigmoid(gi[:, dh:2 * dh] + gh[:, dh:2 * dh])
    nn = jnp.tanh(gi[:, 2 * dh:] + r * gh[:, 2 * dh:])
    o_ref[...] = (1.0 - z) * nn + z * h


def _gru(partials, degp, cb, h, wih, whh, bih, bhh):
    n, dh = h.shape
    d3 = wih.shape[1]
    blk = 2000
    return pl.pallas_call(
        _gru_body,
        grid=(n // blk,),
        in_specs=[
            pl.BlockSpec((2, blk, dh), lambda i: (0, i, 0)),
            pl.BlockSpec((2, blk, dh), lambda i: (0, i, 0)),
            pl.BlockSpec((1, dh), lambda i: (0, 0)),
            pl.BlockSpec((blk, dh), lambda i: (i, 0)),
            pl.BlockSpec((dh, d3), lambda i: (0, 0)),
            pl.BlockSpec((dh, d3), lambda i: (0, 0)),
            pl.BlockSpec((1, d3), lambda i: (0, 0)),
            pl.BlockSpec((1, d3), lambda i: (0, 0)),
        ],
        out_specs=pl.BlockSpec((blk, dh), lambda i: (i, 0)),
        out_shape=jax.ShapeDtypeStruct((n, dh), jnp.float32),
    )(partials, degp, cb, h, wih, whh, bih, bhh)


def _s2s_body(out_ref, batch_ref, wih_ref, whh_ref, bih_ref, bhh_ref,
              l1w_ref, l1b_ref, l2w_ref, l2b_ref, y_ref):
    hi = lax.Precision.HIGHEST
    xo = out_ref[...]
    n, dh = xo.shape
    rows = lax.broadcasted_iota(jnp.int32, (_B, n), 0)
    mb = jnp.broadcast_to(batch_ref[...], (_B, n))
    mask = rows == mb

    q_star = jnp.zeros((_B, 2 * dh), jnp.float32)
    hh = jnp.zeros((_B, dh), jnp.float32)
    cc = jnp.zeros((_B, dh), jnp.float32)
    for _ in range(3):
        g = (jnp.dot(q_star, wih_ref[...], preferred_element_type=jnp.float32,
                     precision=hi) + bih_ref[...] +
             jnp.dot(hh, whh_ref[...], preferred_element_type=jnp.float32,
                     precision=hi) + bhh_ref[...])
        ig = jax.nn.sigmoid(g[:, :dh])
        fg = jax.nn.sigmoid(g[:, dh:2 * dh])
        cg = jnp.tanh(g[:, 2 * dh:3 * dh])
        og = jax.nn.sigmoid(g[:, 3 * dh:])
        cc = fg * cc + ig * cg
        hh = og * jnp.tanh(cc)
        e_all = lax.dot_general(hh, xo, (((1,), (1,)), ((), ())),
                                preferred_element_type=jnp.float32,
                                precision=hi)
        e_masked = jnp.where(mask, e_all, -jnp.inf)
        emax = jnp.max(e_masked, axis=1, keepdims=True)
        emax = jnp.where(jnp.isfinite(emax), emax, 0.0)
        a = jnp.where(mask, jnp.exp(e_all - emax), 0.0)
        denom = jnp.sum(a, axis=1, keepdims=True)
        a = a / jnp.maximum(denom, 1e-30)
        r_read = jnp.dot(a, xo, preferred_element_type=jnp.float32,
                         precision=hi)
        q_star = jnp.concatenate([hh, r_read], axis=1)
    y = jax.nn.relu(
        jnp.dot(q_star, l1w_ref[...], preferred_element_type=jnp.float32,
                precision=hi) + l1b_ref[...])
    y_ref[...] = jnp.dot(y, l2w_ref[...], preferred_element_type=jnp.float32,
                         precision=hi) + l2b_ref[...]


def _s2s(out, batch2d, wih, whh, bih, bhh, l1w, l1b, l2w, l2b):
    dout = l2w.shape[1]
    return pl.pallas_call(
        _s2s_body,
        out_shape=jax.ShapeDtypeStruct((_B, dout), jnp.float32),
    )(out, batch2d, wih, whh, bih, bhh, l1w, l1b, l2w, l2b)


def kernel(x, edge_index, edge_attr, batch, lin0_W, lin0_b, edge_W1, edge_b1,
           edge_W2, edge_b2, conv_b, gru_Wih, gru_Whh, gru_bih, gru_bhh,
           lstm_Wih, lstm_Whh, lstm_bih, lstm_bhh, lin1_W, lin1_b, lin2_W,
           lin2_b):
    n = x.shape[0]
    dh = lin0_W.shape[1]
    d2 = dh * dh
    nlayers = edge_W1.shape[0]
    src = edge_index[0]
    dst = edge_index[1]

    j = jnp.arange(d2, dtype=jnp.int32)
    qm = (j[None, :] // dh == jnp.arange(dh, dtype=jnp.int32)[:, None]
          ).astype(jnp.bfloat16)
    rm = (j[:, None] % dh == jnp.arange(dh, dtype=jnp.int32)[None, :]
          ).astype(jnp.bfloat16)

    h = _lin0(x, lin0_W, lin0_b.reshape(1, -1))
    degp = None
    for li in range(nlayers):
        out_src = _sc_gather(h, src)
        msg = _edge_msgs(edge_attr, out_src, edge_W1[li],
                         edge_b1[li].reshape(1, -1), edge_W2[li],
                         edge_b2[li].reshape(1, -1), qm, rm)
        if degp is None:
            partials, degp = _sc_scatter(msg, dst, n, True)
        else:
            partials = _sc_scatter(msg, dst, n, False)
        h = _gru(partials, degp, conv_b[li].reshape(1, -1), h, gru_Wih,
                 gru_Whh, gru_bih.reshape(1, -1), gru_bhh.reshape(1, -1))
    return _s2s(h, batch.reshape(1, -1), lstm_Wih, lstm_Whh,
                lstm_bih.reshape(1, -1), lstm_bhh.reshape(1, -1), lin1_W,
                lin1_b.reshape(1, -1), lin2_W, lin2_b.reshape(1, -1))
